# baseline (speedup 1.0000x reference)
import functools
import math

import jax
import jax.numpy as jnp
from jax import lax
from jax.experimental import pallas as pl
from jax.experimental.pallas import tpu as pltpu
from jax.experimental.pallas import tpu_sc as plsc

N = 10000
E = 320000
B = 32
L = 32
LD = 256
H = 128
C = 18
OBB = 3 + C

BLK = 1000
CHUNK = 128
NWORK = 32
NROW = 2
NIDX = 4
UNROLL = 4
CHUNKS_PER_W = 80
STEPS = CHUNKS_PER_W // UNROLL
E_PAD = CHUNKS_PER_W * NWORK * CHUNK
ROWS_PER_TILE = 626
N_PAD = 16 * ROWS_PER_TILE
TRASH = N_PAD - 8


def _make_sc_scatter(F):
  mesh = plsc.VectorSubcoreMesh(core_axis_name="c", subcore_axis_name="s")

  @functools.partial(
      pl.kernel,
      out_type=jax.ShapeDtypeStruct((2 * N_PAD, F), jnp.float32),
      mesh=mesh,
      compiler_params=pltpu.CompilerParams(use_tc_tiling_on_sc=False),
      scratch_types=[pltpu.VMEM((2, CHUNK), jnp.int32) for _ in range(NIDX)]
        + [pltpu.VMEM((CHUNK, F), jnp.float32) for _ in range(NROW)]
        + [pltpu.SemaphoreType.DMA for _ in range(NIDX + 2 * NROW)]
        + [pltpu.VMEM_SHARED((N_PAD, F), jnp.float32)],
  )
  def k(tab_hbm, idx_hbm, zeros_hbm, out_hbm, *rest):
    idx = rest[:NIDX]
    rows = rest[NIDX:NIDX + NROW]
    sem_i = rest[NIDX + NROW:2 * NIDX + NROW]
    sem_g = rest[2 * NIDX + NROW:2 * NIDX + 2 * NROW]
    sem_s = rest[2 * NIDX + 2 * NROW:2 * NIDX + 3 * NROW]
    acc_sh = rest[2 * NIDX + 3 * NROW]
    cid = lax.axis_index("c")
    sid = lax.axis_index("s")
    wid = sid * 2 + cid
    c0 = wid * CHUNKS_PER_W
    r0 = sid * ROWS_PER_TILE

    def idx_copy(c, d):
      return pltpu.make_async_copy(idx_hbm.at[c0 + c], idx[d], sem_i[d])

    def gat_copy(c, d, b):
      return pltpu.make_async_copy(tab_hbm.at[idx[d].at[0]], rows[b],
                                   sem_g[b])

    def sca_copy(c, d, b):
      return pltpu.make_async_copy(rows[b], acc_sh.at[idx[d].at[1]],
                                   sem_s[b])

    idx_copy(0, 0).start()
    pltpu.sync_copy(zeros_hbm.at[pl.ds(r0, ROWS_PER_TILE)],
                    acc_sh.at[pl.ds(r0, ROWS_PER_TILE)])
    plsc.subcore_barrier()

    def step(kk, carry):
      for u in range(UNROLL):
        i = kk * UNROLL + u
        b = u % NROW
        d = u % NIDX
        idx_copy(i, d).wait()
        gat_copy(i, d, b).start()

        @pl.when(i + 1 < CHUNKS_PER_W)
        def _():
          idx_copy(i + 1, (u + 1) % NIDX).start()

        gat_copy(i, d, b).wait()
        pltpu.async_copy(rows[b], acc_sh.at[idx[d].at[1]], sem_s[b],
                         add=True)
        sca_copy(i, d, b).wait()

      return carry

    lax.fori_loop(0, STEPS, step, 0)
    plsc.subcore_barrier()
    pltpu.sync_copy(acc_sh.at[pl.ds(r0, ROWS_PER_TILE)],
                    out_hbm.at[pl.ds(cid * N_PAD + r0, ROWS_PER_TILE)])

  return k


def _lang_body(lf_ref, wl1, bl1, bng, bnb, wl2, bl2, out_ref):
  x = jnp.dot(lf_ref[...], wl1[...], preferred_element_type=jnp.float32)
  x = x + bl1[...]
  mu = jnp.mean(x, axis=0, keepdims=True)
  var = jnp.mean((x - mu) ** 2, axis=0, keepdims=True)
  x = (x - mu) * jax.lax.rsqrt(var + 1e-5) * bng[...] + bnb[...]
  x = jnp.maximum(x, 0.0)
  out_ref[...] = jnp.dot(x, wl2[...], preferred_element_type=jnp.float32) \
      + bl2[...]


def _vis_body(pts_ref, wv1, bv1, lng, lnb, wv2, bv2, out_ref):
  v = jnp.dot(pts_ref[...], wv1[...], preferred_element_type=jnp.float32)
  v = v + bv1[...]
  m = jnp.mean(v, axis=-1, keepdims=True)
  s = jnp.mean((v - m) ** 2, axis=-1, keepdims=True)
  v = (v - m) * jax.lax.rsqrt(s + 1e-5) * lng[...] + lnb[...]
  v = jnp.maximum(v, 0.0)
  out_ref[...] = jnp.dot(v, wv2[...], preferred_element_type=jnp.float32) \
      + bv2[...]


def _layer_body(is_last, g_ref, sga, sgb, s0a, s0b, xyz, obb, bidx, lemb,
                mbias, wag, wao, wmg, wmo, wmr, bm, wc, bc, wf1, bf1, wf2,
                bf2, out_ref):
  g = g_ref[...]
  xq = jnp.dot(g, wag[...], preferred_element_type=jnp.float32) + \
      jnp.dot(obb[...], wao[...], preferred_element_type=jnp.float32)
  sc = lax.dot_general(xq, lemb[...], (((1,), (1,)), ((), ())),
                       preferred_element_type=jnp.float32)
  sc = sc * (1.0 / math.sqrt(float(H)))
  colb = lax.broadcasted_iota(jnp.int32, (BLK, B * L), 1) // L
  in_batch = colb == bidx[...]
  sc = jnp.where(in_batch, sc + mbias[...], -3e9)
  mx = jnp.max(sc, axis=-1, keepdims=True)
  e = jnp.exp(sc - mx)
  attn = e / jnp.sum(e, axis=-1, keepdims=True)
  ctx = jnp.dot(attn, lemb[...], preferred_element_type=jnp.float32)
  s0 = s0a[...] + s0b[...]
  sobb = s0[:, :OBB]
  sxyz = s0[:, OBB:OBB + 3]
  deg = s0[:, OBB + 3:OBB + 4]
  srel = deg * xyz[...] - sxyz
  degc = jnp.maximum(deg, 1.0)
  sg = sga[...] + sgb[...]
  agg = (jnp.dot(sg, wmg[...], preferred_element_type=jnp.float32)
         + jnp.dot(sobb, wmo[...], preferred_element_type=jnp.float32)
         + jnp.dot(srel, wmr[...], preferred_element_type=jnp.float32)
         + deg * bm[...]) / degc
  out = agg + jnp.dot(ctx, wc[...], preferred_element_type=jnp.float32) \
      + bc[...]
  out = jnp.maximum(out, 0.0)
  if is_last:
    h1 = jnp.dot(out, wf1[...], preferred_element_type=jnp.float32) + bf1[...]
    h1 = jnp.maximum(h1, 0.0)
    s = jnp.dot(h1, wf2[...], preferred_element_type=jnp.float32) + bf2[...]
    out_ref[...] = jax.nn.sigmoid(s)
  else:
    out_ref[...] = out


def _row_spec(cols):
  return pl.BlockSpec((BLK, cols), lambda i: (i, 0))


def _const_spec(shape):
  return pl.BlockSpec(shape, lambda i: (0, 0))


def _layer_call(is_last, *args):
  grid = (N // BLK,)
  in_specs = [
      _row_spec(H),
      _row_spec(H), _row_spec(H),
      _row_spec(32), _row_spec(32),
      _row_spec(3), _row_spec(OBB), _row_spec(1),
      _const_spec((B * L, H)), _const_spec((1, B * L)),
      _const_spec((H, H)), _const_spec((OBB, H)),
      _const_spec((H, H)), _const_spec((OBB, H)), _const_spec((3, H)),
      _const_spec((1, H)),
      _const_spec((H, H)), _const_spec((1, H)),
      _const_spec((H, H // 2)), _const_spec((1, H // 2)),
      _const_spec((H // 2, 1)), _const_spec((1, 1)),
  ]
  ocols = 1 if is_last else H
  return pl.pallas_call(
      functools.partial(_layer_body, is_last),
      grid=grid,
      in_specs=in_specs,
      out_specs=_row_spec(ocols),
      out_shape=jax.ShapeDtypeStruct((N, ocols), jnp.float32),
  )(*args)


def kernel(pts_feat, obb_feat, support_xyz, lang_feats, lang_mask,
           edge_index, batch_index, params):
  p = params
  f32 = jnp.float32
  padw = (E_PAD - E) // NWORK
  pad_src = jnp.zeros((NWORK, padw), jnp.int32)
  pad_dst = jnp.broadcast_to(N + (jnp.arange(padw, dtype=jnp.int32)
                                  % (N_PAD - N)), (NWORK, padw))
  src = jnp.concatenate([edge_index[0].reshape(NWORK, E // NWORK),
                         pad_src], axis=1)
  dst = jnp.concatenate([edge_index[1].reshape(NWORK, E // NWORK),
                         pad_dst], axis=1)
  idxp = jnp.stack([src.reshape(E_PAD // CHUNK, CHUNK),
                    dst.reshape(E_PAD // CHUNK, CHUNK)], axis=1)

  lang_emb = pl.pallas_call(
      _lang_body,
      out_shape=jax.ShapeDtypeStruct((B * L, H), f32),
  )(lang_feats.reshape(B * L, LD), p['W_l1'], p['b_l1'].reshape(1, H),
    p['bn_g'].reshape(1, H), p['bn_b'].reshape(1, H), p['W_l2'],
    p['b_l2'].reshape(1, H))

  v = pl.pallas_call(
      _vis_body,
      grid=(N // BLK,),
      in_specs=[_row_spec(128)] + [_const_spec(s) for s in
                                   [(128, H), (1, H), (1, H), (1, H),
                                    (H, H), (1, H)]],
      out_specs=_row_spec(H),
      out_shape=jax.ShapeDtypeStruct((N, H), f32),
  )(pts_feat, p['W_v1'], p['b_v1'].reshape(1, H), p['ln_g'].reshape(1, H),
    p['ln_b'].reshape(1, H), p['W_v2'], p['b_v2'].reshape(1, H))

  t0 = jnp.concatenate([obb_feat, support_xyz, jnp.ones((N, 1), f32),
                        jnp.zeros((N, 7), f32)], axis=-1)
  zeros32 = jnp.zeros((N_PAD, 32), f32)
  zeros128 = jnp.zeros((N_PAD, H), f32)
  scat32 = _make_sc_scatter(32)
  scat128 = _make_sc_scatter(H)

  s0 = scat32(t0, idxp, zeros32)
  s0a, s0b = s0[:N], s0[N_PAD:N_PAD + N]

  mbias = jnp.where(lang_mask.reshape(1, B * L) > 0, 0.0, -1e9).astype(f32)
  bidx = batch_index.reshape(N, 1)

  g = v
  for i in (1, 2, 3):
    sg = scat128(g, idxp, zeros128)
    wa = p['Wa%d' % i]
    wm = p['Wm%d' % i]
    g = _layer_call(
        i == 3, g, sg[:N], sg[N_PAD:N_PAD + N], s0a, s0b, support_xyz,
        obb_feat, bidx,
        lang_emb, mbias, wa[:H], wa[H:], wm[:H], wm[H:H + OBB],
        wm[H + OBB:], p['bm%d' % i].reshape(1, H), p['Wc%d' % i],
        p['bc%d' % i].reshape(1, H), p['W_f1'], p['b_f1'].reshape(1, H // 2),
        p['W_f2'], p['b_f2'].reshape(1, 1))
  return g.reshape(N)


# speedup vs baseline: 1.0012x; 1.0012x over previous
import functools
import math

import jax
import jax.numpy as jnp
from jax import lax
from jax.experimental import pallas as pl
from jax.experimental.pallas import tpu as pltpu
from jax.experimental.pallas import tpu_sc as plsc

N = 10000
E = 320000
B = 32
L = 32
LD = 256
H = 128
C = 18
OBB = 3 + C

BLK = 1000
CHUNK = 128
NWORK = 32
NROW = 2
NIDX = 4
UNROLL = 4
CHUNKS_PER_W = 80
STEPS = CHUNKS_PER_W // UNROLL
E_PAD = CHUNKS_PER_W * NWORK * CHUNK
ROWS_PER_TILE = 626
N_PAD = 16 * ROWS_PER_TILE
TRASH = N_PAD - 8


def _make_sc_scatter(F):
  mesh = plsc.VectorSubcoreMesh(core_axis_name="c", subcore_axis_name="s")

  @functools.partial(
      pl.kernel,
      out_type=jax.ShapeDtypeStruct((2 * N_PAD, F), jnp.float32),
      mesh=mesh,
      compiler_params=pltpu.CompilerParams(use_tc_tiling_on_sc=False),
      scratch_types=[pltpu.VMEM((2, CHUNK), jnp.int32) for _ in range(NIDX)]
        + [pltpu.VMEM((CHUNK, F), jnp.float32) for _ in range(NROW)]
        + [pltpu.SemaphoreType.DMA for _ in range(NIDX + 2 * NROW)]
        + [pltpu.VMEM_SHARED((N_PAD, F), jnp.float32)],
  )
  def k(tab_hbm, idx_hbm, zeros_hbm, out_hbm, *rest):
    idx = rest[:NIDX]
    rows = rest[NIDX:NIDX + NROW]
    sem_i = rest[NIDX + NROW:2 * NIDX + NROW]
    sem_g = rest[2 * NIDX + NROW:2 * NIDX + 2 * NROW]
    sem_s = rest[2 * NIDX + 2 * NROW:2 * NIDX + 3 * NROW]
    acc_sh = rest[2 * NIDX + 3 * NROW]
    cid = lax.axis_index("c")
    sid = lax.axis_index("s")
    wid = sid * 2 + cid
    c0 = wid * CHUNKS_PER_W
    r0 = sid * ROWS_PER_TILE

    def idx_copy(c, d):
      return pltpu.make_async_copy(idx_hbm.at[c0 + c], idx[d], sem_i[d])

    def gat_copy(c, d, b):
      return pltpu.make_async_copy(tab_hbm.at[idx[d].at[0]], rows[b],
                                   sem_g[b])

    def sca_copy(c, d, b):
      return pltpu.make_async_copy(rows[b], acc_sh.at[idx[d].at[1]],
                                   sem_s[b])

    idx_copy(0, 0).start()
    pltpu.sync_copy(zeros_hbm.at[pl.ds(r0, ROWS_PER_TILE)],
                    acc_sh.at[pl.ds(r0, ROWS_PER_TILE)])
    plsc.subcore_barrier()

    def step(kk, carry):
      for u in range(UNROLL):
        i = kk * UNROLL + u
        b = u % NROW
        d = u % NIDX
        idx_copy(i, d).wait()

        @pl.when(i + 1 < CHUNKS_PER_W)
        def _():
          idx_copy(i + 1, (u + 1) % NIDX).start()

        pltpu.async_copy(tab_hbm.at[idx[d].at[0]], rows[b], sem_g[b]).wait()
        pltpu.sync_copy(rows[b], acc_sh.at[idx[d].at[1]], add=True)

      return carry

    lax.fori_loop(0, STEPS, step, 0)
    plsc.subcore_barrier()
    pltpu.sync_copy(acc_sh.at[pl.ds(r0, ROWS_PER_TILE)],
                    out_hbm.at[pl.ds(cid * N_PAD + r0, ROWS_PER_TILE)])

  return k


def _lang_body(lf_ref, wl1, bl1, bng, bnb, wl2, bl2, out_ref):
  x = jnp.dot(lf_ref[...], wl1[...], preferred_element_type=jnp.float32)
  x = x + bl1[...]
  mu = jnp.mean(x, axis=0, keepdims=True)
  var = jnp.mean((x - mu) ** 2, axis=0, keepdims=True)
  x = (x - mu) * jax.lax.rsqrt(var + 1e-5) * bng[...] + bnb[...]
  x = jnp.maximum(x, 0.0)
  out_ref[...] = jnp.dot(x, wl2[...], preferred_element_type=jnp.float32) \
      + bl2[...]


def _vis_body(pts_ref, wv1, bv1, lng, lnb, wv2, bv2, out_ref):
  v = jnp.dot(pts_ref[...], wv1[...], preferred_element_type=jnp.float32)
  v = v + bv1[...]
  m = jnp.mean(v, axis=-1, keepdims=True)
  s = jnp.mean((v - m) ** 2, axis=-1, keepdims=True)
  v = (v - m) * jax.lax.rsqrt(s + 1e-5) * lng[...] + lnb[...]
  v = jnp.maximum(v, 0.0)
  out_ref[...] = jnp.dot(v, wv2[...], preferred_element_type=jnp.float32) \
      + bv2[...]


def _layer_body(is_last, g_ref, sga, sgb, s0a, s0b, xyz, obb, bidx, lemb,
                mbias, wag, wao, wmg, wmo, wmr, bm, wc, bc, wf1, bf1, wf2,
                bf2, out_ref):
  g = g_ref[...]
  xq = jnp.dot(g, wag[...], preferred_element_type=jnp.float32) + \
      jnp.dot(obb[...], wao[...], preferred_element_type=jnp.float32)
  sc = lax.dot_general(xq, lemb[...], (((1,), (1,)), ((), ())),
                       preferred_element_type=jnp.float32)
  sc = sc * (1.0 / math.sqrt(float(H)))
  colb = lax.broadcasted_iota(jnp.int32, (BLK, B * L), 1) // L
  in_batch = colb == bidx[...]
  sc = jnp.where(in_batch, sc + mbias[...], -3e9)
  mx = jnp.max(sc, axis=-1, keepdims=True)
  e = jnp.exp(sc - mx)
  attn = e / jnp.sum(e, axis=-1, keepdims=True)
  ctx = jnp.dot(attn, lemb[...], preferred_element_type=jnp.float32)
  s0 = s0a[...] + s0b[...]
  sobb = s0[:, :OBB]
  sxyz = s0[:, OBB:OBB + 3]
  deg = s0[:, OBB + 3:OBB + 4]
  srel = deg * xyz[...] - sxyz
  degc = jnp.maximum(deg, 1.0)
  sg = sga[...] + sgb[...]
  agg = (jnp.dot(sg, wmg[...], preferred_element_type=jnp.float32)
         + jnp.dot(sobb, wmo[...], preferred_element_type=jnp.float32)
         + jnp.dot(srel, wmr[...], preferred_element_type=jnp.float32)
         + deg * bm[...]) / degc
  out = agg + jnp.dot(ctx, wc[...], preferred_element_type=jnp.float32) \
      + bc[...]
  out = jnp.maximum(out, 0.0)
  if is_last:
    h1 = jnp.dot(out, wf1[...], preferred_element_type=jnp.float32) + bf1[...]
    h1 = jnp.maximum(h1, 0.0)
    s = jnp.dot(h1, wf2[...], preferred_element_type=jnp.float32) + bf2[...]
    out_ref[...] = jax.nn.sigmoid(s)
  else:
    out_ref[...] = out


def _row_spec(cols):
  return pl.BlockSpec((BLK, cols), lambda i: (i, 0))


def _const_spec(shape):
  return pl.BlockSpec(shape, lambda i: (0, 0))


def _layer_call(is_last, *args):
  grid = (N // BLK,)
  in_specs = [
      _row_spec(H),
      _row_spec(H), _row_spec(H),
      _row_spec(32), _row_spec(32),
      _row_spec(3), _row_spec(OBB), _row_spec(1),
      _const_spec((B * L, H)), _const_spec((1, B * L)),
      _const_spec((H, H)), _const_spec((OBB, H)),
      _const_spec((H, H)), _const_spec((OBB, H)), _const_spec((3, H)),
      _const_spec((1, H)),
      _const_spec((H, H)), _const_spec((1, H)),
      _const_spec((H, H // 2)), _const_spec((1, H // 2)),
      _const_spec((H // 2, 1)), _const_spec((1, 1)),
  ]
  ocols = 1 if is_last else H
  return pl.pallas_call(
      functools.partial(_layer_body, is_last),
      grid=grid,
      in_specs=in_specs,
      out_specs=_row_spec(ocols),
      out_shape=jax.ShapeDtypeStruct((N, ocols), jnp.float32),
  )(*args)


def kernel(pts_feat, obb_feat, support_xyz, lang_feats, lang_mask,
           edge_index, batch_index, params):
  p = params
  f32 = jnp.float32
  padw = (E_PAD - E) // NWORK
  pad_src = jnp.zeros((NWORK, padw), jnp.int32)
  pad_dst = jnp.broadcast_to(N + (jnp.arange(padw, dtype=jnp.int32)
                                  % (N_PAD - N)), (NWORK, padw))
  src = jnp.concatenate([edge_index[0].reshape(NWORK, E // NWORK),
                         pad_src], axis=1)
  dst = jnp.concatenate([edge_index[1].reshape(NWORK, E // NWORK),
                         pad_dst], axis=1)
  idxp = jnp.stack([src.reshape(E_PAD // CHUNK, CHUNK),
                    dst.reshape(E_PAD // CHUNK, CHUNK)], axis=1)

  lang_emb = pl.pallas_call(
      _lang_body,
      out_shape=jax.ShapeDtypeStruct((B * L, H), f32),
  )(lang_feats.reshape(B * L, LD), p['W_l1'], p['b_l1'].reshape(1, H),
    p['bn_g'].reshape(1, H), p['bn_b'].reshape(1, H), p['W_l2'],
    p['b_l2'].reshape(1, H))

  v = pl.pallas_call(
      _vis_body,
      grid=(N // BLK,),
      in_specs=[_row_spec(128)] + [_const_spec(s) for s in
                                   [(128, H), (1, H), (1, H), (1, H),
                                    (H, H), (1, H)]],
      out_specs=_row_spec(H),
      out_shape=jax.ShapeDtypeStruct((N, H), f32),
  )(pts_feat, p['W_v1'], p['b_v1'].reshape(1, H), p['ln_g'].reshape(1, H),
    p['ln_b'].reshape(1, H), p['W_v2'], p['b_v2'].reshape(1, H))

  t0 = jnp.concatenate([obb_feat, support_xyz, jnp.ones((N, 1), f32),
                        jnp.zeros((N, 7), f32)], axis=-1)
  zeros32 = jnp.zeros((N_PAD, 32), f32)
  zeros128 = jnp.zeros((N_PAD, H), f32)
  scat32 = _make_sc_scatter(32)
  scat128 = _make_sc_scatter(H)

  s0 = scat32(t0, idxp, zeros32)
  s0a, s0b = s0[:N], s0[N_PAD:N_PAD + N]

  mbias = jnp.where(lang_mask.reshape(1, B * L) > 0, 0.0, -1e9).astype(f32)
  bidx = batch_index.reshape(N, 1)

  g = v
  for i in (1, 2, 3):
    sg = scat128(g, idxp, zeros128)
    wa = p['Wa%d' % i]
    wm = p['Wm%d' % i]
    g = _layer_call(
        i == 3, g, sg[:N], sg[N_PAD:N_PAD + N], s0a, s0b, support_xyz,
        obb_feat, bidx,
        lang_emb, mbias, wa[:H], wa[H:], wm[:H], wm[H:H + OBB],
        wm[H + OBB:], p['bm%d' % i].reshape(1, H), p['Wc%d' % i],
        p['bc%d' % i].reshape(1, H), p['W_f1'], p['b_f1'].reshape(1, H // 2),
        p['W_f2'], p['b_f2'].reshape(1, 1))
  return g.reshape(N)


# speedup vs baseline: 1.1821x; 1.1807x over previous
import functools
import math

import jax
import jax.numpy as jnp
from jax import lax
from jax.experimental import pallas as pl
from jax.experimental.pallas import tpu as pltpu
from jax.experimental.pallas import tpu_sc as plsc

N = 10000
E = 320000
B = 32
L = 32
LD = 256
H = 128
C = 18
OBB = 3 + C

BLK = 1000
CHUNK = 128
NWORK = 32
NROW = 2
NIDX = 4
UNROLL = 4
CHUNKS_PER_W = 80
STEPS = CHUNKS_PER_W // UNROLL
E_PAD = CHUNKS_PER_W * NWORK * CHUNK
ROWS_PER_TILE = 626
N_PAD = 16 * ROWS_PER_TILE
TRASH = N_PAD - 8


def _make_sc_scatter(F):
  mesh = plsc.VectorSubcoreMesh(core_axis_name="c", subcore_axis_name="s")

  @functools.partial(
      pl.kernel,
      out_type=jax.ShapeDtypeStruct((2 * N_PAD, F), jnp.float32),
      mesh=mesh,
      compiler_params=pltpu.CompilerParams(use_tc_tiling_on_sc=False),
      scratch_types=[pltpu.VMEM((CHUNK,), jnp.int32) for _ in range(4)]
        + [pltpu.VMEM((CHUNK, F), jnp.float32)]
        + [pltpu.SemaphoreType.DMA for _ in range(3)]
        + [pltpu.VMEM_SHARED((N_PAD, F), jnp.float32)],
  )
  def k(tab_hbm, src_hbm, dst_hbm, zeros_hbm, out_hbm, *rest):
    srcv = rest[0:2]
    dstv = rest[2:4]
    rows = rest[4]
    sem_i = rest[5:7]
    sem_g = rest[7]
    acc_sh = rest[8]
    cid = lax.axis_index("c")
    sid = lax.axis_index("s")
    wid = sid * 2 + cid
    r0 = sid * ROWS_PER_TILE

    def idx_copies(i, d):
      base = (wid + NWORK * i) * CHUNK
      return (pltpu.make_async_copy(src_hbm.at[pl.ds(base, CHUNK)], srcv[d],
                                    sem_i[d]),
              pltpu.make_async_copy(dst_hbm.at[pl.ds(base, CHUNK)], dstv[d],
                                    sem_i[d]))

    for c in idx_copies(0, 0):
      c.start()
    pltpu.sync_copy(zeros_hbm.at[pl.ds(r0, ROWS_PER_TILE)],
                    acc_sh.at[pl.ds(r0, ROWS_PER_TILE)])
    plsc.subcore_barrier()

    def step(kk, carry):
      for u in range(2):
        i = kk * 2 + u
        d = u % 2
        for c in idx_copies(i, d):
          c.wait()

        @pl.when(i + 1 < CHUNKS_PER_W)
        def _():
          for c in idx_copies(i + 1, 1 - d):
            c.start()

        pltpu.async_copy(tab_hbm.at[srcv[d]], rows, sem_g).wait()
        pltpu.sync_copy(rows, acc_sh.at[dstv[d]], add=True)

      return carry

    lax.fori_loop(0, CHUNKS_PER_W // 2, step, 0)
    plsc.subcore_barrier()
    pltpu.sync_copy(acc_sh.at[pl.ds(r0, ROWS_PER_TILE)],
                    out_hbm.at[pl.ds(cid * N_PAD + r0, ROWS_PER_TILE)])

  return k


def _lang_body(lf_ref, wl1, bl1, bng, bnb, wl2, bl2, out_ref):
  x = jnp.dot(lf_ref[...], wl1[...], preferred_element_type=jnp.float32)
  x = x + bl1[...]
  mu = jnp.mean(x, axis=0, keepdims=True)
  var = jnp.mean((x - mu) ** 2, axis=0, keepdims=True)
  x = (x - mu) * jax.lax.rsqrt(var + 1e-5) * bng[...] + bnb[...]
  x = jnp.maximum(x, 0.0)
  out_ref[...] = jnp.dot(x, wl2[...], preferred_element_type=jnp.float32) \
      + bl2[...]


def _vis_body(pts_ref, wv1, bv1, lng, lnb, wv2, bv2, out_ref):
  v = jnp.dot(pts_ref[...], wv1[...], preferred_element_type=jnp.float32)
  v = v + bv1[...]
  m = jnp.mean(v, axis=-1, keepdims=True)
  s = jnp.mean((v - m) ** 2, axis=-1, keepdims=True)
  v = (v - m) * jax.lax.rsqrt(s + 1e-5) * lng[...] + lnb[...]
  v = jnp.maximum(v, 0.0)
  out_ref[...] = jnp.dot(v, wv2[...], preferred_element_type=jnp.float32) \
      + bv2[...]


def _layer_body(is_last, g_ref, sga, sgb, s0a, s0b, xyz, obb, bidx, lemb,
                mbias, wag, wao, wmg, wmo, wmr, bm, wc, bc, wf1, bf1, wf2,
                bf2, out_ref):
  g = g_ref[...]
  xq = jnp.dot(g, wag[...], preferred_element_type=jnp.float32) + \
      jnp.dot(obb[...], wao[...], preferred_element_type=jnp.float32)
  sc = lax.dot_general(xq, lemb[...], (((1,), (1,)), ((), ())),
                       preferred_element_type=jnp.float32)
  sc = sc * (1.0 / math.sqrt(float(H)))
  colb = lax.broadcasted_iota(jnp.int32, (BLK, B * L), 1) // L
  in_batch = colb == bidx[...]
  sc = jnp.where(in_batch, sc + mbias[...], -3e9)
  mx = jnp.max(sc, axis=-1, keepdims=True)
  e = jnp.exp(sc - mx)
  attn = e / jnp.sum(e, axis=-1, keepdims=True)
  ctx = jnp.dot(attn, lemb[...], preferred_element_type=jnp.float32)
  s0 = s0a[...] + s0b[...]
  sobb = s0[:, :OBB]
  sxyz = s0[:, OBB:OBB + 3]
  deg = s0[:, OBB + 3:OBB + 4]
  srel = deg * xyz[...] - sxyz
  degc = jnp.maximum(deg, 1.0)
  sg = sga[...] + sgb[...]
  agg = (jnp.dot(sg, wmg[...], preferred_element_type=jnp.float32)
         + jnp.dot(sobb, wmo[...], preferred_element_type=jnp.float32)
         + jnp.dot(srel, wmr[...], preferred_element_type=jnp.float32)
         + deg * bm[...]) / degc
  out = agg + jnp.dot(ctx, wc[...], preferred_element_type=jnp.float32) \
      + bc[...]
  out = jnp.maximum(out, 0.0)
  if is_last:
    h1 = jnp.dot(out, wf1[...], preferred_element_type=jnp.float32) + bf1[...]
    h1 = jnp.maximum(h1, 0.0)
    s = jnp.dot(h1, wf2[...], preferred_element_type=jnp.float32) + bf2[...]
    out_ref[...] = jax.nn.sigmoid(s)
  else:
    out_ref[...] = out


def _row_spec(cols):
  return pl.BlockSpec((BLK, cols), lambda i: (i, 0))


def _const_spec(shape):
  return pl.BlockSpec(shape, lambda i: (0, 0))


def _layer_call(is_last, *args):
  grid = (N // BLK,)
  in_specs = [
      _row_spec(H),
      _row_spec(H), _row_spec(H),
      _row_spec(32), _row_spec(32),
      _row_spec(3), _row_spec(OBB), _row_spec(1),
      _const_spec((B * L, H)), _const_spec((1, B * L)),
      _const_spec((H, H)), _const_spec((OBB, H)),
      _const_spec((H, H)), _const_spec((OBB, H)), _const_spec((3, H)),
      _const_spec((1, H)),
      _const_spec((H, H)), _const_spec((1, H)),
      _const_spec((H, H // 2)), _const_spec((1, H // 2)),
      _const_spec((H // 2, 1)), _const_spec((1, 1)),
  ]
  ocols = 1 if is_last else H
  return pl.pallas_call(
      functools.partial(_layer_body, is_last),
      grid=grid,
      in_specs=in_specs,
      out_specs=_row_spec(ocols),
      out_shape=jax.ShapeDtypeStruct((N, ocols), jnp.float32),
  )(*args)


def kernel(pts_feat, obb_feat, support_xyz, lang_feats, lang_mask,
           edge_index, batch_index, params):
  p = params
  f32 = jnp.float32
  padw = (E_PAD - E) // NWORK
  pad_src = jnp.zeros((NWORK, padw), jnp.int32)
  pad_dst = jnp.broadcast_to(N + (jnp.arange(padw, dtype=jnp.int32)
                                  % (N_PAD - N)), (NWORK, padw))
  src = jnp.concatenate([edge_index[0].reshape(NWORK, E // NWORK),
                         pad_src], axis=1)
  dst = jnp.concatenate([edge_index[1].reshape(NWORK, E // NWORK),
                         pad_dst], axis=1)
  src = src.reshape(E_PAD)
  dst = dst.reshape(E_PAD)

  lang_emb = pl.pallas_call(
      _lang_body,
      out_shape=jax.ShapeDtypeStruct((B * L, H), f32),
  )(lang_feats.reshape(B * L, LD), p['W_l1'], p['b_l1'].reshape(1, H),
    p['bn_g'].reshape(1, H), p['bn_b'].reshape(1, H), p['W_l2'],
    p['b_l2'].reshape(1, H))

  v = pl.pallas_call(
      _vis_body,
      grid=(N // BLK,),
      in_specs=[_row_spec(128)] + [_const_spec(s) for s in
                                   [(128, H), (1, H), (1, H), (1, H),
                                    (H, H), (1, H)]],
      out_specs=_row_spec(H),
      out_shape=jax.ShapeDtypeStruct((N, H), f32),
  )(pts_feat, p['W_v1'], p['b_v1'].reshape(1, H), p['ln_g'].reshape(1, H),
    p['ln_b'].reshape(1, H), p['W_v2'], p['b_v2'].reshape(1, H))

  t0 = jnp.concatenate([obb_feat, support_xyz, jnp.ones((N, 1), f32),
                        jnp.zeros((N, 7), f32)], axis=-1)
  zeros32 = jnp.zeros((N_PAD, 32), f32)
  zeros128 = jnp.zeros((N_PAD, H), f32)
  scat32 = _make_sc_scatter(32)
  scat128 = _make_sc_scatter(H)

  s0 = scat32(t0, src, dst, zeros32)
  s0a, s0b = s0[:N], s0[N_PAD:N_PAD + N]

  mbias = jnp.where(lang_mask.reshape(1, B * L) > 0, 0.0, -1e9).astype(f32)
  bidx = batch_index.reshape(N, 1)

  g = v
  for i in (1, 2, 3):
    sg = scat128(g, src, dst, zeros128)
    wa = p['Wa%d' % i]
    wm = p['Wm%d' % i]
    g = _layer_call(
        i == 3, g, sg[:N], sg[N_PAD:N_PAD + N], s0a, s0b, support_xyz,
        obb_feat, bidx,
        lang_emb, mbias, wa[:H], wa[H:], wm[:H], wm[H:H + OBB],
        wm[H + OBB:], p['bm%d' % i].reshape(1, H), p['Wc%d' % i],
        p['bc%d' % i].reshape(1, H), p['W_f1'], p['b_f1'].reshape(1, H // 2),
        p['W_f2'], p['b_f2'].reshape(1, 1))
  return g.reshape(N)


# speedup vs baseline: 1.2052x; 1.0195x over previous
import functools
import math

import jax
import jax.numpy as jnp
from jax import lax
from jax.experimental import pallas as pl
from jax.experimental.pallas import tpu as pltpu
from jax.experimental.pallas import tpu_sc as plsc

N = 10000
E = 320000
B = 32
L = 32
LD = 256
H = 128
C = 18
OBB = 3 + C

BLK = 1000
CHUNK = 128
NWORK = 32
CHUNKS_PER_W = 80
E_PAD = CHUNKS_PER_W * NWORK * CHUNK
ROWS_PER_TILE = 626
N_PAD = 16 * ROWS_PER_TILE
F1 = 160


def _make_sc_scatter(F):
  mesh = plsc.VectorSubcoreMesh(core_axis_name="c", subcore_axis_name="s")

  @functools.partial(
      pl.kernel,
      out_type=jax.ShapeDtypeStruct((2 * N_PAD, F), jnp.float32),
      mesh=mesh,
      compiler_params=pltpu.CompilerParams(use_tc_tiling_on_sc=False),
      scratch_types=[
          pltpu.VMEM((CHUNK,), jnp.int32),
          pltpu.VMEM((CHUNK,), jnp.int32),
          pltpu.VMEM((CHUNK, F), jnp.float32),
          pltpu.SemaphoreType.DMA,
          pltpu.VMEM_SHARED((N_PAD, F), jnp.float32),
      ],
  )
  def k(tab_hbm, src_hbm, dst_hbm, zeros_hbm, out_hbm, srcv, dstv, rows,
        sem_g, acc_sh):
    cid = lax.axis_index("c")
    sid = lax.axis_index("s")
    wid = sid * 2 + cid
    r0 = sid * ROWS_PER_TILE
    pltpu.sync_copy(zeros_hbm.at[pl.ds(r0, ROWS_PER_TILE)],
                    acc_sh.at[pl.ds(r0, ROWS_PER_TILE)])
    plsc.subcore_barrier()

    def step(i, carry):
      base = (wid + NWORK * i) * CHUNK
      pltpu.sync_copy(src_hbm.at[pl.ds(base, CHUNK)], srcv)
      pltpu.sync_copy(dst_hbm.at[pl.ds(base, CHUNK)], dstv)
      pltpu.async_copy(tab_hbm.at[srcv], rows, sem_g).wait()
      pltpu.sync_copy(rows, acc_sh.at[dstv], add=True)
      return carry

    lax.fori_loop(0, CHUNKS_PER_W, step, 0)
    plsc.subcore_barrier()
    pltpu.sync_copy(acc_sh.at[pl.ds(r0, ROWS_PER_TILE)],
                    out_hbm.at[pl.ds(cid * N_PAD + r0, ROWS_PER_TILE)])

  return k


def _lang_body(lf_ref, wl1, bl1, bng, bnb, wl2, bl2, out_ref):
  x = jnp.dot(lf_ref[...], wl1[...], preferred_element_type=jnp.float32)
  x = x + bl1[...]
  mu = jnp.mean(x, axis=0, keepdims=True)
  var = jnp.mean((x - mu) ** 2, axis=0, keepdims=True)
  x = (x - mu) * jax.lax.rsqrt(var + 1e-5) * bng[...] + bnb[...]
  x = jnp.maximum(x, 0.0)
  out_ref[...] = jnp.dot(x, wl2[...], preferred_element_type=jnp.float32) \
      + bl2[...]


def _vis_body(pts_ref, wv1, bv1, lng, lnb, wv2, bv2, out_ref):
  v = jnp.dot(pts_ref[...], wv1[...], preferred_element_type=jnp.float32)
  v = v + bv1[...]
  m = jnp.mean(v, axis=-1, keepdims=True)
  s = jnp.mean((v - m) ** 2, axis=-1, keepdims=True)
  v = (v - m) * jax.lax.rsqrt(s + 1e-5) * lng[...] + lnb[...]
  v = jnp.maximum(v, 0.0)
  out_ref[...] = jnp.dot(v, wv2[...], preferred_element_type=jnp.float32) \
      + bv2[...]


def _attn_body(g_ref, obb, bidx, lemb, mbias, wag, wao, wc, bc, out_ref):
  g = g_ref[...]
  xq = jnp.dot(g, wag[...], preferred_element_type=jnp.float32) + \
      jnp.dot(obb[...], wao[...], preferred_element_type=jnp.float32)
  sc = lax.dot_general(xq, lemb[...], (((1,), (1,)), ((), ())),
                       preferred_element_type=jnp.float32)
  sc = sc * (1.0 / math.sqrt(float(H)))
  colb = lax.broadcasted_iota(jnp.int32, (BLK, B * L), 1) // L
  in_batch = colb == bidx[...]
  sc = jnp.where(in_batch, sc + mbias[...], -3e9)
  mx = jnp.max(sc, axis=-1, keepdims=True)
  e = jnp.exp(sc - mx)
  attn = e / jnp.sum(e, axis=-1, keepdims=True)
  ctx = jnp.dot(attn, lemb[...], preferred_element_type=jnp.float32)
  out_ref[...] = jnp.dot(ctx, wc[...], preferred_element_type=jnp.float32) \
      + bc[...]


def _comb_body(is_last, ctxt, sga, sgb, s0a, s0b, xyz, wmg, wmo, wmr, bm,
               wf1, bf1, wf2, bf2, out_ref):
  s0 = s0a[...] + s0b[...]
  sobb = s0[:, :OBB]
  sxyz = s0[:, OBB:OBB + 3]
  deg = s0[:, OBB + 3:OBB + 4]
  srel = deg * xyz[...] - sxyz
  degc = jnp.maximum(deg, 1.0)
  sg = sga[...] + sgb[...]
  agg = (jnp.dot(sg, wmg[...], preferred_element_type=jnp.float32)
         + jnp.dot(sobb, wmo[...], preferred_element_type=jnp.float32)
         + jnp.dot(srel, wmr[...], preferred_element_type=jnp.float32)
         + deg * bm[...]) / degc
  out = jnp.maximum(agg + ctxt[...], 0.0)
  if is_last:
    h1 = jnp.dot(out, wf1[...], preferred_element_type=jnp.float32) + bf1[...]
    h1 = jnp.maximum(h1, 0.0)
    s = jnp.dot(h1, wf2[...], preferred_element_type=jnp.float32) + bf2[...]
    out_ref[...] = jax.nn.sigmoid(s)
  else:
    out_ref[...] = out


def _row_spec(cols):
  return pl.BlockSpec((BLK, cols), lambda i: (i, 0))


def _const_spec(shape):
  return pl.BlockSpec(shape, lambda i: (0, 0))


def _attn_call(g, obb, bidx, lemb, mbias, wag, wao, wc, bc):
  in_specs = [
      _row_spec(H), _row_spec(OBB), _row_spec(1),
      _const_spec((B * L, H)), _const_spec((1, B * L)),
      _const_spec((H, H)), _const_spec((OBB, H)),
      _const_spec((H, H)), _const_spec((1, H)),
  ]
  return pl.pallas_call(
      _attn_body,
      grid=(N // BLK,),
      in_specs=in_specs,
      out_specs=_row_spec(H),
      out_shape=jax.ShapeDtypeStruct((N, H), jnp.float32),
  )(g, obb, bidx, lemb, mbias, wag, wao, wc, bc)


def _comb_call(is_last, *args):
  in_specs = [
      _row_spec(H),
      _row_spec(H), _row_spec(H),
      _row_spec(32), _row_spec(32),
      _row_spec(3),
      _const_spec((H, H)), _const_spec((OBB, H)), _const_spec((3, H)),
      _const_spec((1, H)),
      _const_spec((H, H // 2)), _const_spec((1, H // 2)),
      _const_spec((H // 2, 1)), _const_spec((1, 1)),
  ]
  ocols = 1 if is_last else H
  return pl.pallas_call(
      functools.partial(_comb_body, is_last),
      grid=(N // BLK,),
      in_specs=in_specs,
      out_specs=_row_spec(ocols),
      out_shape=jax.ShapeDtypeStruct((N, ocols), jnp.float32),
  )(*args)


def kernel(pts_feat, obb_feat, support_xyz, lang_feats, lang_mask,
           edge_index, batch_index, params):
  p = params
  f32 = jnp.float32
  padw = (E_PAD - E) // NWORK
  pad_src = jnp.zeros((NWORK, padw), jnp.int32)
  pad_dst = jnp.broadcast_to(N + (jnp.arange(padw, dtype=jnp.int32)
                                  % (N_PAD - N)), (NWORK, padw))
  src = jnp.concatenate([edge_index[0].reshape(NWORK, E // NWORK),
                         pad_src], axis=1)
  dst = jnp.concatenate([edge_index[1].reshape(NWORK, E // NWORK),
                         pad_dst], axis=1)
  src = src.reshape(E_PAD)
  dst = dst.reshape(E_PAD)

  lang_emb = pl.pallas_call(
      _lang_body,
      out_shape=jax.ShapeDtypeStruct((B * L, H), f32),
  )(lang_feats.reshape(B * L, LD), p['W_l1'], p['b_l1'].reshape(1, H),
    p['bn_g'].reshape(1, H), p['bn_b'].reshape(1, H), p['W_l2'],
    p['b_l2'].reshape(1, H))

  v = pl.pallas_call(
      _vis_body,
      grid=(N // BLK,),
      in_specs=[_row_spec(128)] + [_const_spec(s) for s in
                                   [(128, H), (1, H), (1, H), (1, H),
                                    (H, H), (1, H)]],
      out_specs=_row_spec(H),
      out_shape=jax.ShapeDtypeStruct((N, H), f32),
  )(pts_feat, p['W_v1'], p['b_v1'].reshape(1, H), p['ln_g'].reshape(1, H),
    p['ln_b'].reshape(1, H), p['W_v2'], p['b_v2'].reshape(1, H))

  t1 = jnp.concatenate([v, obb_feat, support_xyz, jnp.ones((N, 1), f32),
                        jnp.zeros((N, F1 - H - OBB - 4), f32)], axis=-1)
  zeros160 = jnp.zeros((N_PAD, F1), f32)
  zeros128 = jnp.zeros((N_PAD, H), f32)
  scat160 = _make_sc_scatter(F1)
  scat128 = _make_sc_scatter(H)

  s1 = scat160(t1, src, dst, zeros160)
  s0a, s0b = s1[:N, H:], s1[N_PAD:N_PAD + N, H:]
  sga, sgb = s1[:N, :H], s1[N_PAD:N_PAD + N, :H]

  mbias = jnp.where(lang_mask.reshape(1, B * L) > 0, 0.0, -1e9).astype(f32)
  bidx = batch_index.reshape(N, 1)

  g = v
  for i in (1, 2, 3):
    wa = p['Wa%d' % i]
    wm = p['Wm%d' % i]
    ctxt = _attn_call(g, obb_feat, bidx, lang_emb, mbias, wa[:H], wa[H:],
                      p['Wc%d' % i], p['bc%d' % i].reshape(1, H))
    g = _comb_call(
        i == 3, ctxt, sga, sgb, s0a, s0b, support_xyz, wm[:H],
        wm[H:H + OBB], wm[H + OBB:], p['bm%d' % i].reshape(1, H),
        p['W_f1'], p['b_f1'].reshape(1, H // 2), p['W_f2'],
        p['b_f2'].reshape(1, 1))
    if i < 3:
      sg = scat128(g, src, dst, zeros128)
      sga, sgb = sg[:N], sg[N_PAD:N_PAD + N]
  return g.reshape(N)


# speedup vs baseline: 1.8119x; 1.5034x over previous
import functools
import math

import jax
import jax.numpy as jnp
from jax import lax
from jax.experimental import pallas as pl
from jax.experimental.pallas import tpu as pltpu
from jax.experimental.pallas import tpu_sc as plsc

N = 10000
E = 320000
B = 32
L = 32
LD = 256
H = 128
C = 18
OBB = 3 + C

BLK = 1000
CHUNK = 128
NWORK = 32
NCHUNK = E // CHUNK
PERW = (NCHUNK + NWORK - 1) // NWORK
ROWS_PER_TILE = 626
N_PAD = 16 * ROWS_PER_TILE
F1 = 160


def _make_sc_scatter(F):
  mesh = plsc.VectorSubcoreMesh(core_axis_name="c", subcore_axis_name="s")

  @functools.partial(
      pl.kernel,
      out_type=jax.ShapeDtypeStruct((2 * N_PAD, F), jnp.float32),
      mesh=mesh,
      compiler_params=pltpu.CompilerParams(use_tc_tiling_on_sc=False),
      scratch_types=[
          pltpu.VMEM((CHUNK,), jnp.int32),
          pltpu.VMEM((CHUNK,), jnp.int32),
          pltpu.VMEM((CHUNK, F), jnp.float32),
          pltpu.SemaphoreType.DMA,
          pltpu.VMEM_SHARED((N_PAD, F), jnp.float32),
      ],
  )
  def k(tab_hbm, src_hbm, dst_hbm, zeros_hbm, out_hbm, srcv, dstv, rows,
        sem_g, acc_sh):
    cid = lax.axis_index("c")
    sid = lax.axis_index("s")
    wid = sid * 2 + cid
    r0 = sid * ROWS_PER_TILE
    pltpu.sync_copy(zeros_hbm.at[pl.ds(r0, ROWS_PER_TILE)],
                    acc_sh.at[pl.ds(r0, ROWS_PER_TILE)])
    plsc.subcore_barrier()

    def step(i, carry):
      c = wid + NWORK * i

      @pl.when(c < NCHUNK)
      def _():
        base = c * CHUNK
        pltpu.sync_copy(src_hbm.at[pl.ds(base, CHUNK)], srcv)
        pltpu.sync_copy(dst_hbm.at[pl.ds(base, CHUNK)], dstv)
        pltpu.async_copy(tab_hbm.at[srcv], rows, sem_g).wait()
        pltpu.sync_copy(rows, acc_sh.at[dstv], add=True)

      return carry

    lax.fori_loop(0, PERW, step, 0)
    plsc.subcore_barrier()
    pltpu.sync_copy(acc_sh.at[pl.ds(r0, ROWS_PER_TILE)],
                    out_hbm.at[pl.ds(cid * N_PAD + r0, ROWS_PER_TILE)])

  return k


def _lang_body(lf_ref, wl1, bl1, bng, bnb, wl2, bl2, out_ref):
  x = jnp.dot(lf_ref[...], wl1[...], preferred_element_type=jnp.float32)
  x = x + bl1[...]
  mu = jnp.mean(x, axis=0, keepdims=True)
  var = jnp.mean((x - mu) ** 2, axis=0, keepdims=True)
  x = (x - mu) * jax.lax.rsqrt(var + 1e-5) * bng[...] + bnb[...]
  x = jnp.maximum(x, 0.0)
  out_ref[...] = jnp.dot(x, wl2[...], preferred_element_type=jnp.float32) \
      + bl2[...]


def _vis_body(pts_ref, wv1, bv1, lng, lnb, wv2, bv2, out_ref):
  v = jnp.dot(pts_ref[...], wv1[...], preferred_element_type=jnp.float32)
  v = v + bv1[...]
  m = jnp.mean(v, axis=-1, keepdims=True)
  s = jnp.mean((v - m) ** 2, axis=-1, keepdims=True)
  v = (v - m) * jax.lax.rsqrt(s + 1e-5) * lng[...] + lnb[...]
  v = jnp.maximum(v, 0.0)
  out_ref[...] = jnp.dot(v, wv2[...], preferred_element_type=jnp.float32) \
      + bv2[...]


def _attn_body(g_ref, obb, bidx, lemb, mbias, wag, wao, wc, bc, out_ref):
  g = g_ref[...]
  xq = jnp.dot(g, wag[...], preferred_element_type=jnp.float32) + \
      jnp.dot(obb[...], wao[...], preferred_element_type=jnp.float32)
  sc = lax.dot_general(xq, lemb[...], (((1,), (1,)), ((), ())),
                       preferred_element_type=jnp.float32)
  sc = sc * (1.0 / math.sqrt(float(H)))
  colb = lax.broadcasted_iota(jnp.int32, (BLK, B * L), 1) // L
  in_batch = colb == bidx[...]
  sc = jnp.where(in_batch, sc + mbias[...], -3e9)
  mx = jnp.max(sc, axis=-1, keepdims=True)
  e = jnp.exp(sc - mx)
  attn = e / jnp.sum(e, axis=-1, keepdims=True)
  ctx = jnp.dot(attn, lemb[...], preferred_element_type=jnp.float32)
  out_ref[...] = jnp.dot(ctx, wc[...], preferred_element_type=jnp.float32) \
      + bc[...]


def _comb_body(is_last, ctxt, sga, sgb, s0a, s0b, xyz, wmg, wmo, wmr, bm,
               wf1, bf1, wf2, bf2, out_ref):
  s0 = s0a[...] + s0b[...]
  sobb = s0[:, :OBB]
  sxyz = s0[:, OBB:OBB + 3]
  deg = s0[:, OBB + 3:OBB + 4]
  srel = deg * xyz[...] - sxyz
  degc = jnp.maximum(deg, 1.0)
  sg = sga[...] + sgb[...]
  agg = (jnp.dot(sg, wmg[...], preferred_element_type=jnp.float32)
         + jnp.dot(sobb, wmo[...], preferred_element_type=jnp.float32)
         + jnp.dot(srel, wmr[...], preferred_element_type=jnp.float32)
         + deg * bm[...]) / degc
  out = jnp.maximum(agg + ctxt[...], 0.0)
  if is_last:
    h1 = jnp.dot(out, wf1[...], preferred_element_type=jnp.float32) + bf1[...]
    h1 = jnp.maximum(h1, 0.0)
    s = jnp.dot(h1, wf2[...], preferred_element_type=jnp.float32) + bf2[...]
    out_ref[...] = jax.nn.sigmoid(s)
  else:
    out_ref[...] = out


def _row_spec(cols):
  return pl.BlockSpec((BLK, cols), lambda i: (i, 0))


def _const_spec(shape):
  return pl.BlockSpec(shape, lambda i: (0, 0))


def _attn_call(g, obb, bidx, lemb, mbias, wag, wao, wc, bc):
  in_specs = [
      _row_spec(H), _row_spec(OBB), _row_spec(1),
      _const_spec((B * L, H)), _const_spec((1, B * L)),
      _const_spec((H, H)), _const_spec((OBB, H)),
      _const_spec((H, H)), _const_spec((1, H)),
  ]
  return pl.pallas_call(
      _attn_body,
      grid=(N // BLK,),
      in_specs=in_specs,
      out_specs=_row_spec(H),
      out_shape=jax.ShapeDtypeStruct((N, H), jnp.float32),
  )(g, obb, bidx, lemb, mbias, wag, wao, wc, bc)


def _comb_call(is_last, *args):
  in_specs = [
      _row_spec(H),
      _row_spec(H), _row_spec(H),
      _row_spec(32), _row_spec(32),
      _row_spec(3),
      _const_spec((H, H)), _const_spec((OBB, H)), _const_spec((3, H)),
      _const_spec((1, H)),
      _const_spec((H, H // 2)), _const_spec((1, H // 2)),
      _const_spec((H // 2, 1)), _const_spec((1, 1)),
  ]
  ocols = 1 if is_last else H
  return pl.pallas_call(
      functools.partial(_comb_body, is_last),
      grid=(N // BLK,),
      in_specs=in_specs,
      out_specs=_row_spec(ocols),
      out_shape=jax.ShapeDtypeStruct((N, ocols), jnp.float32),
  )(*args)


def kernel(pts_feat, obb_feat, support_xyz, lang_feats, lang_mask,
           edge_index, batch_index, params):
  p = params
  f32 = jnp.float32
  src = edge_index[0]
  dst = edge_index[1]

  lang_emb = pl.pallas_call(
      _lang_body,
      out_shape=jax.ShapeDtypeStruct((B * L, H), f32),
  )(lang_feats.reshape(B * L, LD), p['W_l1'], p['b_l1'].reshape(1, H),
    p['bn_g'].reshape(1, H), p['bn_b'].reshape(1, H), p['W_l2'],
    p['b_l2'].reshape(1, H))

  v = pl.pallas_call(
      _vis_body,
      grid=(N // BLK,),
      in_specs=[_row_spec(128)] + [_const_spec(s) for s in
                                   [(128, H), (1, H), (1, H), (1, H),
                                    (H, H), (1, H)]],
      out_specs=_row_spec(H),
      out_shape=jax.ShapeDtypeStruct((N, H), f32),
  )(pts_feat, p['W_v1'], p['b_v1'].reshape(1, H), p['ln_g'].reshape(1, H),
    p['ln_b'].reshape(1, H), p['W_v2'], p['b_v2'].reshape(1, H))

  t1 = jnp.concatenate([v, obb_feat, support_xyz, jnp.ones((N, 1), f32),
                        jnp.zeros((N, F1 - H - OBB - 4), f32)], axis=-1)
  zeros160 = jnp.zeros((N_PAD, F1), f32)
  zeros128 = jnp.zeros((N_PAD, H), f32)
  scat160 = _make_sc_scatter(F1)
  scat128 = _make_sc_scatter(H)

  s1 = scat160(t1, src, dst, zeros160)
  s0a, s0b = s1[:N, H:], s1[N_PAD:N_PAD + N, H:]
  sga, sgb = s1[:N, :H], s1[N_PAD:N_PAD + N, :H]

  mbias = jnp.where(lang_mask.reshape(1, B * L) > 0, 0.0, -1e9).astype(f32)
  bidx = batch_index.reshape(N, 1)

  g = v
  for i in (1, 2, 3):
    wa = p['Wa%d' % i]
    wm = p['Wm%d' % i]
    ctxt = _attn_call(g, obb_feat, bidx, lang_emb, mbias, wa[:H], wa[H:],
                      p['Wc%d' % i], p['bc%d' % i].reshape(1, H))
    g = _comb_call(
        i == 3, ctxt, sga, sgb, s0a, s0b, support_xyz, wm[:H],
        wm[H:H + OBB], wm[H + OBB:], p['bm%d' % i].reshape(1, H),
        p['W_f1'], p['b_f1'].reshape(1, H // 2), p['W_f2'],
        p['b_f2'].reshape(1, 1))
    if i < 3:
      sg = scat128(g, src, dst, zeros128)
      sga, sgb = sg[:N], sg[N_PAD:N_PAD + N]
  return g.reshape(N)


# speedup vs baseline: 2.5606x; 1.4132x over previous
import functools
import math

import jax
import jax.numpy as jnp
from jax import lax
from jax.experimental import pallas as pl
from jax.experimental.pallas import tpu as pltpu
from jax.experimental.pallas import tpu_sc as plsc

N = 10000
E = 320000
B = 32
L = 32
LD = 256
H = 128
C = 18
OBB = 3 + C

BLK = 1000
CHUNK = 128
NWORK = 32
NCHUNK = E // CHUNK
PERW = (NCHUNK + NWORK - 1) // NWORK
ROWS_PER_TILE = 626
N_PAD = 16 * ROWS_PER_TILE
F1 = 160


def _make_sc_scatter(F):
  mesh = plsc.VectorSubcoreMesh(core_axis_name="c", subcore_axis_name="s")

  @functools.partial(
      pl.kernel,
      out_type=jax.ShapeDtypeStruct((2 * N_PAD, F), jnp.float32),
      mesh=mesh,
      compiler_params=pltpu.CompilerParams(use_tc_tiling_on_sc=False),
      scratch_types=[
          pltpu.VMEM((CHUNK,), jnp.int32),
          pltpu.VMEM((CHUNK,), jnp.int32),
          pltpu.VMEM((CHUNK, F), jnp.float32),
          pltpu.SemaphoreType.DMA,
          pltpu.VMEM_SHARED((N_PAD, F), jnp.float32),
      ],
  )
  def k(tab_hbm, src_hbm, dst_hbm, zeros_hbm, out_hbm, srcv, dstv, rows,
        sem_g, acc_sh):
    cid = lax.axis_index("c")
    sid = lax.axis_index("s")
    wid = sid * 2 + cid
    r0 = sid * ROWS_PER_TILE
    pltpu.sync_copy(zeros_hbm.at[pl.ds(r0, ROWS_PER_TILE)],
                    acc_sh.at[pl.ds(r0, ROWS_PER_TILE)])
    plsc.subcore_barrier()

    def step(i, carry):
      c = wid + NWORK * i

      @pl.when(c < NCHUNK)
      def _():
        base = c * CHUNK
        pltpu.sync_copy(src_hbm.at[pl.ds(base, CHUNK)], srcv)
        pltpu.sync_copy(dst_hbm.at[pl.ds(base, CHUNK)], dstv)
        pltpu.async_copy(tab_hbm.at[srcv], rows, sem_g).wait()
        pltpu.sync_copy(rows, acc_sh.at[dstv], add=True)

      return carry

    lax.fori_loop(0, PERW, step, 0)
    plsc.subcore_barrier()
    pltpu.sync_copy(acc_sh.at[pl.ds(r0, ROWS_PER_TILE)],
                    out_hbm.at[pl.ds(cid * N_PAD + r0, ROWS_PER_TILE)])

  return k


def _make_sc_scatter_pipe(F):
  mesh = plsc.VectorSubcoreMesh(core_axis_name="c", subcore_axis_name="s")

  @functools.partial(
      pl.kernel,
      out_type=jax.ShapeDtypeStruct((2 * N_PAD, F), jnp.float32),
      mesh=mesh,
      compiler_params=pltpu.CompilerParams(use_tc_tiling_on_sc=False),
      scratch_types=[pltpu.VMEM((CHUNK,), jnp.int32) for _ in range(8)]
        + [pltpu.VMEM((CHUNK, F), jnp.float32) for _ in range(2)]
        + [pltpu.SemaphoreType.DMA for _ in range(8)]
        + [pltpu.VMEM_SHARED((N_PAD, F), jnp.float32)],
  )
  def k(tab_hbm, src_hbm, dst_hbm, zeros_hbm, out_hbm, *rest):
    srcv = rest[0:4]
    dstv = rest[4:8]
    rows = rest[8:10]
    sem_i = rest[10:14]
    sem_g = rest[14:16]
    sem_s = rest[16:18]
    acc_sh = rest[18]
    cid = lax.axis_index("c")
    sid = lax.axis_index("s")
    wid = sid * 2 + cid
    r0 = sid * ROWS_PER_TILE

    def valid(i):
      return (wid + NWORK * i) < NCHUNK

    def idx_copies(i, d):
      base = (wid + NWORK * i) * CHUNK
      return (pltpu.make_async_copy(src_hbm.at[pl.ds(base, CHUNK)], srcv[d],
                                    sem_i[d]),
              pltpu.make_async_copy(dst_hbm.at[pl.ds(base, CHUNK)], dstv[d],
                                    sem_i[d]))

    for cp in idx_copies(0, 0):
      cp.start()
    pltpu.sync_copy(zeros_hbm.at[pl.ds(r0, ROWS_PER_TILE)],
                    acc_sh.at[pl.ds(r0, ROWS_PER_TILE)])
    plsc.subcore_barrier()

    def step(kk, carry):
      for u in range(4):
        i = kk * 4 + u
        b = u % 2
        d = u % 4
        b1 = (u - 1) % 2
        d1 = (u - 1) % 4

        @pl.when(valid(i))
        def _():
          @pl.when(i >= 2)
          def _():
            pltpu.make_async_copy(rows[b], acc_sh.at[dstv[(u - 2) % 4]],
                                  sem_s[b]).wait()

          @pl.when(valid(i + 1))
          def _():
            for cp in idx_copies(i + 1, (u + 1) % 4):
              cp.start()

          for cp in idx_copies(i, d):
            cp.wait()
          pltpu.async_copy(tab_hbm.at[srcv[d]], rows[b], sem_g[b])

        @pl.when((i >= 1) & valid(i - 1))
        def _():
          pltpu.make_async_copy(tab_hbm.at[srcv[d1]], rows[b1],
                                sem_g[b1]).wait()
          pltpu.async_copy(rows[b1], acc_sh.at[dstv[d1]], sem_s[b1],
                           add=True)

      return carry

    lax.fori_loop(0, (PERW + 1) // 4, step, 0)
    for b in range(2):
      pltpu.make_async_copy(rows[b], acc_sh.at[dstv[b]], sem_s[b]).wait()
    plsc.subcore_barrier()
    pltpu.sync_copy(acc_sh.at[pl.ds(r0, ROWS_PER_TILE)],
                    out_hbm.at[pl.ds(cid * N_PAD + r0, ROWS_PER_TILE)])

  return k


def _lang_body(lf_ref, wl1, bl1, bng, bnb, wl2, bl2, out_ref):
  x = jnp.dot(lf_ref[...], wl1[...], preferred_element_type=jnp.float32)
  x = x + bl1[...]
  mu = jnp.mean(x, axis=0, keepdims=True)
  var = jnp.mean((x - mu) ** 2, axis=0, keepdims=True)
  x = (x - mu) * jax.lax.rsqrt(var + 1e-5) * bng[...] + bnb[...]
  x = jnp.maximum(x, 0.0)
  out_ref[...] = jnp.dot(x, wl2[...], preferred_element_type=jnp.float32) \
      + bl2[...]


def _vis_body(pts_ref, wv1, bv1, lng, lnb, wv2, bv2, out_ref):
  v = jnp.dot(pts_ref[...], wv1[...], preferred_element_type=jnp.float32)
  v = v + bv1[...]
  m = jnp.mean(v, axis=-1, keepdims=True)
  s = jnp.mean((v - m) ** 2, axis=-1, keepdims=True)
  v = (v - m) * jax.lax.rsqrt(s + 1e-5) * lng[...] + lnb[...]
  v = jnp.maximum(v, 0.0)
  out_ref[...] = jnp.dot(v, wv2[...], preferred_element_type=jnp.float32) \
      + bv2[...]


def _attn_body(g_ref, obb, bidx, lemb, mbias, wag, wao, wc, bc, out_ref):
  g = g_ref[...]
  xq = jnp.dot(g, wag[...], preferred_element_type=jnp.float32) + \
      jnp.dot(obb[...], wao[...], preferred_element_type=jnp.float32)
  sc = lax.dot_general(xq, lemb[...], (((1,), (1,)), ((), ())),
                       preferred_element_type=jnp.float32)
  sc = sc * (1.0 / math.sqrt(float(H)))
  colb = lax.broadcasted_iota(jnp.int32, (BLK, B * L), 1) // L
  in_batch = colb == bidx[...]
  sc = jnp.where(in_batch, sc + mbias[...], -3e9)
  mx = jnp.max(sc, axis=-1, keepdims=True)
  e = jnp.exp(sc - mx)
  attn = e / jnp.sum(e, axis=-1, keepdims=True)
  ctx = jnp.dot(attn, lemb[...], preferred_element_type=jnp.float32)
  out_ref[...] = jnp.dot(ctx, wc[...], preferred_element_type=jnp.float32) \
      + bc[...]


def _comb_body(is_last, ctxt, sga, sgb, s0a, s0b, xyz, wmg, wmo, wmr, bm,
               wf1, bf1, wf2, bf2, out_ref):
  s0 = s0a[...] + s0b[...]
  sobb = s0[:, :OBB]
  sxyz = s0[:, OBB:OBB + 3]
  deg = s0[:, OBB + 3:OBB + 4]
  srel = deg * xyz[...] - sxyz
  degc = jnp.maximum(deg, 1.0)
  sg = sga[...] + sgb[...]
  agg = (jnp.dot(sg, wmg[...], preferred_element_type=jnp.float32)
         + jnp.dot(sobb, wmo[...], preferred_element_type=jnp.float32)
         + jnp.dot(srel, wmr[...], preferred_element_type=jnp.float32)
         + deg * bm[...]) / degc
  out = jnp.maximum(agg + ctxt[...], 0.0)
  if is_last:
    h1 = jnp.dot(out, wf1[...], preferred_element_type=jnp.float32) + bf1[...]
    h1 = jnp.maximum(h1, 0.0)
    s = jnp.dot(h1, wf2[...], preferred_element_type=jnp.float32) + bf2[...]
    out_ref[...] = jax.nn.sigmoid(s)
  else:
    out_ref[...] = out


def _row_spec(cols):
  return pl.BlockSpec((BLK, cols), lambda i: (i, 0))


def _const_spec(shape):
  return pl.BlockSpec(shape, lambda i: (0, 0))


def _attn_call(g, obb, bidx, lemb, mbias, wag, wao, wc, bc):
  in_specs = [
      _row_spec(H), _row_spec(OBB), _row_spec(1),
      _const_spec((B * L, H)), _const_spec((1, B * L)),
      _const_spec((H, H)), _const_spec((OBB, H)),
      _const_spec((H, H)), _const_spec((1, H)),
  ]
  return pl.pallas_call(
      _attn_body,
      grid=(N // BLK,),
      in_specs=in_specs,
      out_specs=_row_spec(H),
      out_shape=jax.ShapeDtypeStruct((N, H), jnp.float32),
  )(g, obb, bidx, lemb, mbias, wag, wao, wc, bc)


def _comb_call(is_last, *args):
  in_specs = [
      _row_spec(H),
      _row_spec(H), _row_spec(H),
      _row_spec(32), _row_spec(32),
      _row_spec(3),
      _const_spec((H, H)), _const_spec((OBB, H)), _const_spec((3, H)),
      _const_spec((1, H)),
      _const_spec((H, H // 2)), _const_spec((1, H // 2)),
      _const_spec((H // 2, 1)), _const_spec((1, 1)),
  ]
  ocols = 1 if is_last else H
  return pl.pallas_call(
      functools.partial(_comb_body, is_last),
      grid=(N // BLK,),
      in_specs=in_specs,
      out_specs=_row_spec(ocols),
      out_shape=jax.ShapeDtypeStruct((N, ocols), jnp.float32),
  )(*args)


def kernel(pts_feat, obb_feat, support_xyz, lang_feats, lang_mask,
           edge_index, batch_index, params):
  p = params
  f32 = jnp.float32
  src = edge_index[0]
  dst = edge_index[1]

  lang_emb = pl.pallas_call(
      _lang_body,
      out_shape=jax.ShapeDtypeStruct((B * L, H), f32),
  )(lang_feats.reshape(B * L, LD), p['W_l1'], p['b_l1'].reshape(1, H),
    p['bn_g'].reshape(1, H), p['bn_b'].reshape(1, H), p['W_l2'],
    p['b_l2'].reshape(1, H))

  v = pl.pallas_call(
      _vis_body,
      grid=(N // BLK,),
      in_specs=[_row_spec(128)] + [_const_spec(s) for s in
                                   [(128, H), (1, H), (1, H), (1, H),
                                    (H, H), (1, H)]],
      out_specs=_row_spec(H),
      out_shape=jax.ShapeDtypeStruct((N, H), f32),
  )(pts_feat, p['W_v1'], p['b_v1'].reshape(1, H), p['ln_g'].reshape(1, H),
    p['ln_b'].reshape(1, H), p['W_v2'], p['b_v2'].reshape(1, H))

  t1 = jnp.concatenate([v, obb_feat, support_xyz, jnp.ones((N, 1), f32),
                        jnp.zeros((N, F1 - H - OBB - 4), f32)], axis=-1)
  zeros160 = jnp.zeros((N_PAD, F1), f32)
  zeros128 = jnp.zeros((N_PAD, H), f32)
  scat160 = _make_sc_scatter(F1)
  scat128 = _make_sc_scatter_pipe(H)

  s1 = scat160(t1, src, dst, zeros160)
  s0a, s0b = s1[:N, H:], s1[N_PAD:N_PAD + N, H:]
  sga, sgb = s1[:N, :H], s1[N_PAD:N_PAD + N, :H]

  mbias = jnp.where(lang_mask.reshape(1, B * L) > 0, 0.0, -1e9).astype(f32)
  bidx = batch_index.reshape(N, 1)

  g = v
  for i in (1, 2, 3):
    wa = p['Wa%d' % i]
    wm = p['Wm%d' % i]
    ctxt = _attn_call(g, obb_feat, bidx, lang_emb, mbias, wa[:H], wa[H:],
                      p['Wc%d' % i], p['bc%d' % i].reshape(1, H))
    g = _comb_call(
        i == 3, ctxt, sga, sgb, s0a, s0b, support_xyz, wm[:H],
        wm[H:H + OBB], wm[H + OBB:], p['bm%d' % i].reshape(1, H),
        p['W_f1'], p['b_f1'].reshape(1, H // 2), p['W_f2'],
        p['b_f2'].reshape(1, 1))
    if i < 3:
      sg = scat128(g, src, dst, zeros128)
      sga, sgb = sg[:N], sg[N_PAD:N_PAD + N]
  return g.reshape(N)


# speedup vs baseline: 3.4437x; 1.3449x over previous
import functools
import math

import jax
import jax.numpy as jnp
from jax import lax
from jax.experimental import pallas as pl
from jax.experimental.pallas import tpu as pltpu
from jax.experimental.pallas import tpu_sc as plsc

N = 10000
E = 320000
B = 32
L = 32
LD = 256
H = 128
C = 18
OBB = 3 + C

BLK = 1000
CHUNK = 128
NWORK = 32
NCHUNK = E // CHUNK
PERW = (NCHUNK + NWORK - 1) // NWORK
ROWS_PER_TILE = 626
N_PAD = 16 * ROWS_PER_TILE

def _make_sc_scatter_pipe(F):
  mesh = plsc.VectorSubcoreMesh(core_axis_name="c", subcore_axis_name="s")

  @functools.partial(
      pl.kernel,
      out_type=jax.ShapeDtypeStruct((2 * N_PAD, F), jnp.float32),
      mesh=mesh,
      compiler_params=pltpu.CompilerParams(use_tc_tiling_on_sc=False),
      scratch_types=[pltpu.VMEM((CHUNK,), jnp.int32) for _ in range(8)]
        + [pltpu.VMEM((CHUNK, F), jnp.float32) for _ in range(2)]
        + [pltpu.SemaphoreType.DMA for _ in range(8)]
        + [pltpu.VMEM_SHARED((N_PAD, F), jnp.float32)],
  )
  def k(tab_hbm, src_hbm, dst_hbm, zeros_hbm, out_hbm, *rest):
    srcv = rest[0:4]
    dstv = rest[4:8]
    rows = rest[8:10]
    sem_i = rest[10:14]
    sem_g = rest[14:16]
    sem_s = rest[16:18]
    acc_sh = rest[18]
    cid = lax.axis_index("c")
    sid = lax.axis_index("s")
    wid = sid * 2 + cid
    r0 = sid * ROWS_PER_TILE

    def valid(i):
      return (wid + NWORK * i) < NCHUNK

    def idx_copies(i, d):
      base = (wid + NWORK * i) * CHUNK
      return (pltpu.make_async_copy(src_hbm.at[pl.ds(base, CHUNK)], srcv[d],
                                    sem_i[d]),
              pltpu.make_async_copy(dst_hbm.at[pl.ds(base, CHUNK)], dstv[d],
                                    sem_i[d]))

    for cp in idx_copies(0, 0):
      cp.start()
    pltpu.sync_copy(zeros_hbm.at[pl.ds(r0, ROWS_PER_TILE)],
                    acc_sh.at[pl.ds(r0, ROWS_PER_TILE)])
    plsc.subcore_barrier()

    def step(kk, carry):
      for u in range(4):
        i = kk * 4 + u
        b = u % 2
        d = u % 4
        b1 = (u - 1) % 2
        d1 = (u - 1) % 4

        @pl.when(valid(i))
        def _():
          @pl.when(i >= 2)
          def _():
            pltpu.make_async_copy(rows[b], acc_sh.at[dstv[(u - 2) % 4]],
                                  sem_s[b]).wait()

          @pl.when(valid(i + 1))
          def _():
            for cp in idx_copies(i + 1, (u + 1) % 4):
              cp.start()

          for cp in idx_copies(i, d):
            cp.wait()
          pltpu.async_copy(tab_hbm.at[srcv[d]], rows[b], sem_g[b])

        @pl.when((i >= 1) & valid(i - 1))
        def _():
          pltpu.make_async_copy(tab_hbm.at[srcv[d1]], rows[b1],
                                sem_g[b1]).wait()
          pltpu.async_copy(rows[b1], acc_sh.at[dstv[d1]], sem_s[b1],
                           add=True)

      return carry

    lax.fori_loop(0, (PERW + 1) // 4, step, 0)
    for b in range(2):
      pltpu.make_async_copy(rows[b], acc_sh.at[dstv[b]], sem_s[b]).wait()
    plsc.subcore_barrier()
    pltpu.sync_copy(acc_sh.at[pl.ds(r0, ROWS_PER_TILE)],
                    out_hbm.at[pl.ds(cid * N_PAD + r0, ROWS_PER_TILE)])

  return k


def _lang_body(lf_ref, wl1, bl1, bng, bnb, wl2, bl2, out_ref):
  x = jnp.dot(lf_ref[...], wl1[...], preferred_element_type=jnp.float32)
  x = x + bl1[...]
  mu = jnp.mean(x, axis=0, keepdims=True)
  var = jnp.mean((x - mu) ** 2, axis=0, keepdims=True)
  x = (x - mu) * jax.lax.rsqrt(var + 1e-5) * bng[...] + bnb[...]
  x = jnp.maximum(x, 0.0)
  out_ref[...] = jnp.dot(x, wl2[...], preferred_element_type=jnp.float32) \
      + bl2[...]


def _vis_body(pts_ref, wv1, bv1, lng, lnb, wv2, bv2, out_ref):
  v = jnp.dot(pts_ref[...], wv1[...], preferred_element_type=jnp.float32)
  v = v + bv1[...]
  m = jnp.mean(v, axis=-1, keepdims=True)
  s = jnp.mean((v - m) ** 2, axis=-1, keepdims=True)
  v = (v - m) * jax.lax.rsqrt(s + 1e-5) * lng[...] + lnb[...]
  v = jnp.maximum(v, 0.0)
  out_ref[...] = jnp.dot(v, wv2[...], preferred_element_type=jnp.float32) \
      + bv2[...]


def _attn_body(g_ref, obb, bidx, lemb, mbias, wag, wao, wc, bc, out_ref):
  g = g_ref[...]
  xq = jnp.dot(g, wag[...], preferred_element_type=jnp.float32) + \
      jnp.dot(obb[...], wao[...], preferred_element_type=jnp.float32)
  sc = lax.dot_general(xq, lemb[...], (((1,), (1,)), ((), ())),
                       preferred_element_type=jnp.float32)
  sc = sc * (1.0 / math.sqrt(float(H)))
  colb = lax.broadcasted_iota(jnp.int32, (BLK, B * L), 1) // L
  in_batch = colb == bidx[...]
  sc = jnp.where(in_batch, sc + mbias[...], -3e9)
  mx = jnp.max(sc, axis=-1, keepdims=True)
  e = jnp.exp(sc - mx)
  attn = e / jnp.sum(e, axis=-1, keepdims=True)
  ctx = jnp.dot(attn, lemb[...], preferred_element_type=jnp.float32)
  out_ref[...] = jnp.dot(ctx, wc[...], preferred_element_type=jnp.float32) \
      + bc[...]


def _comb_body(is_last, ctxt, sga, sgb, s0a, s0b, xyz, wmg, wmo, wmr, bm,
               wf1, bf1, wf2, bf2, out_ref):
  s0 = s0a[...] + s0b[...]
  sobb = s0[:, :OBB]
  sxyz = s0[:, OBB:OBB + 3]
  deg = s0[:, OBB + 3:OBB + 4]
  srel = deg * xyz[...] - sxyz
  degc = jnp.maximum(deg, 1.0)
  sg = sga[...] + sgb[...]
  agg = (jnp.dot(sg, wmg[...], preferred_element_type=jnp.float32)
         + jnp.dot(sobb, wmo[...], preferred_element_type=jnp.float32)
         + jnp.dot(srel, wmr[...], preferred_element_type=jnp.float32)
         + deg * bm[...]) / degc
  out = jnp.maximum(agg + ctxt[...], 0.0)
  if is_last:
    h1 = jnp.dot(out, wf1[...], preferred_element_type=jnp.float32) + bf1[...]
    h1 = jnp.maximum(h1, 0.0)
    s = jnp.dot(h1, wf2[...], preferred_element_type=jnp.float32) + bf2[...]
    out_ref[...] = jax.nn.sigmoid(s)
  else:
    out_ref[...] = out


def _row_spec(cols):
  return pl.BlockSpec((BLK, cols), lambda i: (i, 0))


def _const_spec(shape):
  return pl.BlockSpec(shape, lambda i: (0, 0))


def _attn_call(g, obb, bidx, lemb, mbias, wag, wao, wc, bc):
  in_specs = [
      _row_spec(H), _row_spec(OBB), _row_spec(1),
      _const_spec((B * L, H)), _const_spec((1, B * L)),
      _const_spec((H, H)), _const_spec((OBB, H)),
      _const_spec((H, H)), _const_spec((1, H)),
  ]
  return pl.pallas_call(
      _attn_body,
      grid=(N // BLK,),
      in_specs=in_specs,
      out_specs=_row_spec(H),
      out_shape=jax.ShapeDtypeStruct((N, H), jnp.float32),
  )(g, obb, bidx, lemb, mbias, wag, wao, wc, bc)


def _comb_call(is_last, *args):
  in_specs = [
      _row_spec(H),
      _row_spec(H), _row_spec(H),
      _row_spec(32), _row_spec(32),
      _row_spec(3),
      _const_spec((H, H)), _const_spec((OBB, H)), _const_spec((3, H)),
      _const_spec((1, H)),
      _const_spec((H, H // 2)), _const_spec((1, H // 2)),
      _const_spec((H // 2, 1)), _const_spec((1, 1)),
  ]
  ocols = 1 if is_last else H
  return pl.pallas_call(
      functools.partial(_comb_body, is_last),
      grid=(N // BLK,),
      in_specs=in_specs,
      out_specs=_row_spec(ocols),
      out_shape=jax.ShapeDtypeStruct((N, ocols), jnp.float32),
  )(*args)


def kernel(pts_feat, obb_feat, support_xyz, lang_feats, lang_mask,
           edge_index, batch_index, params):
  p = params
  f32 = jnp.float32
  src = edge_index[0]
  dst = edge_index[1]

  lang_emb = pl.pallas_call(
      _lang_body,
      out_shape=jax.ShapeDtypeStruct((B * L, H), f32),
  )(lang_feats.reshape(B * L, LD), p['W_l1'], p['b_l1'].reshape(1, H),
    p['bn_g'].reshape(1, H), p['bn_b'].reshape(1, H), p['W_l2'],
    p['b_l2'].reshape(1, H))

  v = pl.pallas_call(
      _vis_body,
      grid=(N // BLK,),
      in_specs=[_row_spec(128)] + [_const_spec(s) for s in
                                   [(128, H), (1, H), (1, H), (1, H),
                                    (H, H), (1, H)]],
      out_specs=_row_spec(H),
      out_shape=jax.ShapeDtypeStruct((N, H), f32),
  )(pts_feat, p['W_v1'], p['b_v1'].reshape(1, H), p['ln_g'].reshape(1, H),
    p['ln_b'].reshape(1, H), p['W_v2'], p['b_v2'].reshape(1, H))

  t0 = jnp.concatenate([obb_feat, support_xyz, jnp.ones((N, 1), f32),
                        jnp.zeros((N, 7), f32)], axis=-1)
  zeros32 = jnp.zeros((N_PAD, 32), f32)
  zeros128 = jnp.zeros((N_PAD, H), f32)
  scat32 = _make_sc_scatter_pipe(32)
  scat128 = _make_sc_scatter_pipe(H)

  sg = scat128(v, src, dst, zeros128)
  sga, sgb = sg[:N], sg[N_PAD:N_PAD + N]
  s0 = scat32(t0, src, dst, zeros32)
  s0a, s0b = s0[:N], s0[N_PAD:N_PAD + N]

  mbias = jnp.where(lang_mask.reshape(1, B * L) > 0, 0.0, -1e9).astype(f32)
  bidx = batch_index.reshape(N, 1)

  g = v
  for i in (1, 2, 3):
    wa = p['Wa%d' % i]
    wm = p['Wm%d' % i]
    ctxt = _attn_call(g, obb_feat, bidx, lang_emb, mbias, wa[:H], wa[H:],
                      p['Wc%d' % i], p['bc%d' % i].reshape(1, H))
    g = _comb_call(
        i == 3, ctxt, sga, sgb, s0a, s0b, support_xyz, wm[:H],
        wm[H:H + OBB], wm[H + OBB:], p['bm%d' % i].reshape(1, H),
        p['W_f1'], p['b_f1'].reshape(1, H // 2), p['W_f2'],
        p['b_f2'].reshape(1, 1))
    if i < 3:
      sg = scat128(g, src, dst, zeros128)
      sga, sgb = sg[:N], sg[N_PAD:N_PAD + N]
  return g.reshape(N)
